# baseline (speedup 1.0000x reference)
import functools
import jax
import jax.numpy as jnp
from jax import lax
from jax.experimental import pallas as pl
from jax.experimental.pallas import tpu as pltpu
from jax.experimental.pallas import tpu_sc as plsc

_NUM_EXPERTS = 8
_TOP_K = 2
_HIDDEN = 2048
_FFN = 5632
_TOKENS = 2048

_B = 256
_S = _TOKENS * _TOP_K
_PMAX = ((_S + _NUM_EXPERTS * (_B - 1) + _B - 1) // _B) * _B
_NBLK = _PMAX // _B
_FBLK = 1408
_HBLK = 1024


_KBLK = 1024
_NK = 2048 // _KBLK


def _gemm1_body(be_ref, xs_ref, w1_ref, w3_ref, a_ref, h1_ref, h3_ref):
    k = pl.program_id(2)
    x = xs_ref[...]
    w1 = w1_ref[0].astype(jnp.bfloat16)
    w3 = w3_ref[0].astype(jnp.bfloat16)
    p1 = jax.lax.dot_general(x, w1, (((1,), (0,)), ((), ())),
                             preferred_element_type=jnp.float32)
    p3 = jax.lax.dot_general(x, w3, (((1,), (0,)), ((), ())),
                             preferred_element_type=jnp.float32)

    @pl.when(k == 0)
    def _first():
        h1_ref[...] = p1
        h3_ref[...] = p3

    @pl.when(k == _NK - 1)
    def _last():
        h1 = h1_ref[...] + p1
        h3 = h3_ref[...] + p3
        a_ref[...] = ((h1 * jax.lax.logistic(h1)) * h3).astype(jnp.bfloat16)


def _gemm2_body(be_ref, a_ref, w2_ref, wp_ref, y_ref):
    a = a_ref[...]
    w2 = w2_ref[0].astype(jnp.bfloat16)
    y = jax.lax.dot_general(a, w2, (((1,), (0,)), ((), ())),
                            preferred_element_type=jnp.float32)
    y_ref[...] = (y * wp_ref[...]).astype(jnp.bfloat16)


def _meta_body(ids_ref, pos_ref, ends_ref):
    iota8 = jax.lax.broadcasted_iota(jnp.int32, (_TOKENS, _NUM_EXPERTS), 1)
    oh1 = (ids_ref[:, 0:1] == iota8).astype(jnp.int32)
    oh2 = (ids_ref[:, 1:2] == iota8).astype(jnp.int32)
    oh = oh1 + oh2
    cum = oh
    sh = 1
    while sh < _TOKENS:
        rolled = pltpu.roll(cum, sh, 0)
        row = jax.lax.broadcasted_iota(jnp.int32, (_TOKENS, _NUM_EXPERTS), 0)
        cum = cum + jnp.where(row >= sh, rolled, 0)
        sh *= 2
    cum_ex = cum - oh
    g = jnp.sum(oh, axis=0, keepdims=True)
    gp = ((g + _B - 1) // _B) * _B
    ends = gp
    sh = 1
    while sh < _NUM_EXPERTS:
        rolled = jnp.where(iota8[:1] >= sh, pltpu.roll(ends, sh, 1), 0)
        ends = ends + rolled
        sh *= 2
    off = ends - gp
    rank1 = jnp.sum(oh1 * cum_ex, axis=1, keepdims=True)
    rank2 = jnp.sum(oh2 * cum_ex, axis=1, keepdims=True)
    base1 = jnp.sum(oh1 * off, axis=1, keepdims=True)
    base2 = jnp.sum(oh2 * off, axis=1, keepdims=True)
    pos_ref[:, 0:1] = base1 + rank1
    pos_ref[:, 1:2] = base2 + rank2
    ends_ref[...] = jnp.broadcast_to(ends, (8, _NUM_EXPERTS))


def _routing_meta(topk_ids):
    return pl.pallas_call(
        _meta_body,
        out_shape=(jax.ShapeDtypeStruct((_TOKENS, 2), jnp.int32),
                   jax.ShapeDtypeStruct((8, _NUM_EXPERTS), jnp.int32)),
    )(topk_ids)


def _grouped_mlp(xs, w1, w3, w2, wp, blk_e):
    nf = _FFN // _FBLK
    nh = _HIDDEN // _HBLK
    act = pl.pallas_call(
        _gemm1_body,
        grid_spec=pltpu.PrefetchScalarGridSpec(
            num_scalar_prefetch=1,
            grid=(nf, _NBLK, _NK),
            in_specs=[
                pl.BlockSpec((_B, _KBLK), lambda j, i, k, be: (i, k)),
                pl.BlockSpec((1, _KBLK, _FBLK), lambda j, i, k, be: (be[i], k, j)),
                pl.BlockSpec((1, _KBLK, _FBLK), lambda j, i, k, be: (be[i], k, j)),
            ],
            out_specs=pl.BlockSpec((_B, _FBLK), lambda j, i, k, be: (i, j)),
            scratch_shapes=[
                pltpu.VMEM((_B, _FBLK), jnp.float32),
                pltpu.VMEM((_B, _FBLK), jnp.float32),
            ],
        ),
        out_shape=jax.ShapeDtypeStruct((_PMAX, _FFN), jnp.bfloat16),
    )(blk_e, xs, w1, w3)

    y = pl.pallas_call(
        _gemm2_body,
        grid_spec=pltpu.PrefetchScalarGridSpec(
            num_scalar_prefetch=1,
            grid=(nh, _NBLK),
            in_specs=[
                pl.BlockSpec((_B, _FFN), lambda h, i, be: (i, 0)),
                pl.BlockSpec((1, _FFN, _HBLK), lambda h, i, be: (be[i], 0, h)),
                pl.BlockSpec((_B, 1), lambda h, i, be: (i, 0)),
            ],
            out_specs=pl.BlockSpec((_B, _HBLK), lambda h, i, be: (i, h)),
        ),
        out_shape=jax.ShapeDtypeStruct((_PMAX, _HIDDEN), jnp.bfloat16),
    )(blk_e, act, w2, wp)
    return y


def kernel(hidden_states, Wg, W1, W2, W3):
    orig_shape = hidden_states.shape
    x = hidden_states.reshape(-1, _HIDDEN)

    router_logits = x @ Wg
    routing_weights = jax.nn.softmax(router_logits, axis=-1)
    iota8 = jnp.arange(_NUM_EXPERTS, dtype=jnp.int32)[None, :]
    v1 = jnp.max(routing_weights, axis=-1, keepdims=True)
    i1 = jnp.min(jnp.where(routing_weights == v1, iota8, _NUM_EXPERTS),
                 axis=-1, keepdims=True)
    p2 = jnp.where(iota8 == i1, -jnp.inf, routing_weights)
    v2 = jnp.max(p2, axis=-1, keepdims=True)
    i2 = jnp.min(jnp.where(p2 == v2, iota8, _NUM_EXPERTS),
                 axis=-1, keepdims=True)
    topk_ids = jnp.concatenate([i1, i2], axis=-1)
    topk_weights = jnp.concatenate([v1, v2], axis=-1)
    topk_weights = topk_weights / jnp.sum(topk_weights, axis=-1, keepdims=True)

    pos2, ends8 = _routing_meta(topk_ids.astype(jnp.int32))
    pos = pos2.reshape(-1)
    ends = ends8[0]
    w_flat = topk_weights.reshape(-1)
    tok_flat = (jnp.arange(_S, dtype=jnp.int32) // _TOP_K)
    tok_p = jnp.zeros((_PMAX,), jnp.int32).at[pos].set(tok_flat)
    w_p = jnp.zeros((_PMAX,), jnp.float32).at[pos].set(w_flat)
    b_starts = jnp.arange(_NBLK, dtype=jnp.int32) * _B
    blk_e = jnp.minimum(
        jnp.sum((ends[None, :] <= b_starts[:, None]).astype(jnp.int32), axis=1),
        _NUM_EXPERTS - 1)

    xs = x.astype(jnp.bfloat16)[tok_p]
    y = _grouped_mlp(xs, W1, W3, W2, w_p.reshape(_PMAX, 1), blk_e)

    out = y[pos].astype(jnp.float32).reshape(_TOKENS, _TOP_K, _HIDDEN).sum(axis=1)
    return out.reshape(orig_shape)


# speedup vs baseline: 1.2169x; 1.2169x over previous
import jax
import jax.numpy as jnp
from jax.experimental import pallas as pl
from jax.experimental.pallas import tpu as pltpu

_NUM_EXPERTS = 8
_TOP_K = 2
_HIDDEN = 2048
_FFN = 5632
_TOKENS = 2048

_B = 256
_S = _TOKENS * _TOP_K
_PMAX = ((_S + _NUM_EXPERTS * (_B - 1) + _B - 1) // _B) * _B
_NBLK = _PMAX // _B
_FBLK = 1408
_HBLK = 1024


def _gemm1_body(be_ref, xs_ref, w1_ref, w3_ref, a_ref):
    x = xs_ref[...]
    w1 = w1_ref[0].astype(jnp.bfloat16)
    w3 = w3_ref[0].astype(jnp.bfloat16)
    h1 = jax.lax.dot_general(x, w1, (((1,), (0,)), ((), ())),
                             preferred_element_type=jnp.float32)
    h3 = jax.lax.dot_general(x, w3, (((1,), (0,)), ((), ())),
                             preferred_element_type=jnp.float32)
    a_ref[...] = ((h1 * jax.lax.logistic(h1)) * h3).astype(jnp.bfloat16)


def _gemm2_body(be_ref, a_ref, w2_ref, wp_ref, y_ref):
    a = a_ref[...]
    w2 = w2_ref[0].astype(jnp.bfloat16)
    y = jax.lax.dot_general(a, w2, (((1,), (0,)), ((), ())),
                            preferred_element_type=jnp.float32)
    y_ref[...] = (y * wp_ref[...]).astype(jnp.bfloat16)


def _meta_body(ids_ref, pos_ref, ends_ref):
    iota8 = jax.lax.broadcasted_iota(jnp.int32, (_TOKENS, _NUM_EXPERTS), 1)
    oh1 = (ids_ref[:, 0:1] == iota8).astype(jnp.int32)
    oh2 = (ids_ref[:, 1:2] == iota8).astype(jnp.int32)
    oh = oh1 + oh2
    cum = oh
    sh = 1
    while sh < _TOKENS:
        rolled = pltpu.roll(cum, sh, 0)
        row = jax.lax.broadcasted_iota(jnp.int32, (_TOKENS, _NUM_EXPERTS), 0)
        cum = cum + jnp.where(row >= sh, rolled, 0)
        sh *= 2
    cum_ex = cum - oh
    g = jnp.sum(oh, axis=0, keepdims=True)
    gp = ((g + _B - 1) // _B) * _B
    ends = gp
    sh = 1
    while sh < _NUM_EXPERTS:
        rolled = jnp.where(iota8[:1] >= sh, pltpu.roll(ends, sh, 1), 0)
        ends = ends + rolled
        sh *= 2
    off = ends - gp
    rank1 = jnp.sum(oh1 * cum_ex, axis=1, keepdims=True)
    rank2 = jnp.sum(oh2 * cum_ex, axis=1, keepdims=True)
    base1 = jnp.sum(oh1 * off, axis=1, keepdims=True)
    base2 = jnp.sum(oh2 * off, axis=1, keepdims=True)
    pos_ref[:, 0:1] = base1 + rank1
    pos_ref[:, 1:2] = base2 + rank2
    ends_ref[...] = jnp.broadcast_to(ends, (8, _NUM_EXPERTS))


def _routing_meta(topk_ids):
    return pl.pallas_call(
        _meta_body,
        out_shape=(jax.ShapeDtypeStruct((_TOKENS, 2), jnp.int32),
                   jax.ShapeDtypeStruct((8, _NUM_EXPERTS), jnp.int32)),
    )(topk_ids)


def _grouped_mlp(xs, w1, w3, w2, wp, blk_e):
    nf = _FFN // _FBLK
    nh = _HIDDEN // _HBLK
    act = pl.pallas_call(
        _gemm1_body,
        grid_spec=pltpu.PrefetchScalarGridSpec(
            num_scalar_prefetch=1,
            grid=(nf, _NBLK),
            in_specs=[
                pl.BlockSpec((_B, _HIDDEN), lambda j, i, be: (i, 0)),
                pl.BlockSpec((1, _HIDDEN, _FBLK), lambda j, i, be: (be[i], 0, j)),
                pl.BlockSpec((1, _HIDDEN, _FBLK), lambda j, i, be: (be[i], 0, j)),
            ],
            out_specs=pl.BlockSpec((_B, _FBLK), lambda j, i, be: (i, j)),
        ),
        out_shape=jax.ShapeDtypeStruct((_PMAX, _FFN), jnp.bfloat16),
    )(blk_e, xs, w1, w3)

    y = pl.pallas_call(
        _gemm2_body,
        grid_spec=pltpu.PrefetchScalarGridSpec(
            num_scalar_prefetch=1,
            grid=(nh, _NBLK),
            in_specs=[
                pl.BlockSpec((_B, _FFN), lambda h, i, be: (i, 0)),
                pl.BlockSpec((1, _FFN, _HBLK), lambda h, i, be: (be[i], 0, h)),
                pl.BlockSpec((_B, 1), lambda h, i, be: (i, 0)),
            ],
            out_specs=pl.BlockSpec((_B, _HBLK), lambda h, i, be: (i, h)),
        ),
        out_shape=jax.ShapeDtypeStruct((_PMAX, _HIDDEN), jnp.bfloat16),
    )(blk_e, act, w2, wp)
    return y


def kernel(hidden_states, Wg, W1, W2, W3):
    orig_shape = hidden_states.shape
    x = hidden_states.reshape(-1, _HIDDEN)

    router_logits = x @ Wg
    routing_weights = jax.nn.softmax(router_logits, axis=-1)
    iota8 = jnp.arange(_NUM_EXPERTS, dtype=jnp.int32)[None, :]
    v1 = jnp.max(routing_weights, axis=-1, keepdims=True)
    i1 = jnp.min(jnp.where(routing_weights == v1, iota8, _NUM_EXPERTS),
                 axis=-1, keepdims=True)
    p2 = jnp.where(iota8 == i1, -jnp.inf, routing_weights)
    v2 = jnp.max(p2, axis=-1, keepdims=True)
    i2 = jnp.min(jnp.where(p2 == v2, iota8, _NUM_EXPERTS),
                 axis=-1, keepdims=True)
    topk_ids = jnp.concatenate([i1, i2], axis=-1)
    topk_weights = jnp.concatenate([v1, v2], axis=-1)
    topk_weights = topk_weights / jnp.sum(topk_weights, axis=-1, keepdims=True)

    pos2, ends8 = _routing_meta(topk_ids.astype(jnp.int32))
    pos = pos2.reshape(-1)
    ends = ends8[0]
    w_flat = topk_weights.reshape(-1)
    tok_flat = (jnp.arange(_S, dtype=jnp.int32) // _TOP_K)
    tok_p = jnp.zeros((_PMAX,), jnp.int32).at[pos].set(tok_flat)
    w_p = jnp.zeros((_PMAX,), jnp.float32).at[pos].set(w_flat)
    b_starts = jnp.arange(_NBLK, dtype=jnp.int32) * _B
    blk_e = jnp.minimum(
        jnp.sum((ends[None, :] <= b_starts[:, None]).astype(jnp.int32), axis=1),
        _NUM_EXPERTS - 1)

    xs = x.astype(jnp.bfloat16)[tok_p]
    y = _grouped_mlp(xs, W1, W3, W2, w_p.reshape(_PMAX, 1), blk_e)

    out = y[pos].astype(jnp.float32).reshape(_TOKENS, _TOP_K, _HIDDEN).sum(axis=1)
    return out.reshape(orig_shape)
